# baseline (speedup 1.0000x reference)
import functools

import jax
import jax.numpy as jnp
from jax import lax
from jax.experimental import pallas as pl
from jax.experimental.pallas import tpu as pltpu
from jax.experimental.pallas import tpu_sc as plsc

_B = 16384
_TB = 2048
_NT = _B // _TB
_NW = 32
_BW = _B // _NW
_EPS = 1e-5

_D_CH, _D_NCH = 32, 16
_C_CH, _C_NCH = 128, 4

_f32 = jnp.float32
_bf16 = jnp.bfloat16


def _pad_cols(arr, cp):
    r, c = arr.shape
    rb = min(r, 2048)
    grid = (r + rb - 1) // rb

    def body(in_ref, out_ref):
        out_ref[:, 0:c] = in_ref[...]
        out_ref[:, c:cp] = jnp.zeros((rb, cp - c), arr.dtype)

    return pl.pallas_call(
        body,
        grid=(grid,),
        in_specs=[pl.BlockSpec((rb, c), lambda i: (i, 0))],
        out_specs=pl.BlockSpec((rb, cp), lambda i: (i, 0)),
        out_shape=jax.ShapeDtypeStruct((r, cp), arr.dtype),
        compiler_params=_SEQ,
    )(arr)


def _transpose_pad(arr_t, cp):
    c, r = arr_t.shape
    rb = 256
    grid = (r + rb - 1) // rb

    def body(in_ref, out_ref):
        out_ref[:, 0:c] = in_ref[...].T
        out_ref[:, c:cp] = jnp.zeros((rb, cp - c), arr_t.dtype)

    return pl.pallas_call(
        body,
        grid=(grid,),
        in_specs=[pl.BlockSpec((c, rb), lambda i: (0, i))],
        out_specs=pl.BlockSpec((rb, cp), lambda i: (i, 0)),
        out_shape=jax.ShapeDtypeStruct((r, cp), arr_t.dtype),
        compiler_params=_SEQ,
    )(arr_t)


def _sc_phase(table, idxv, out, bufs, sems, nch, ch, base):
    handles = [None, None]
    handles[0] = pltpu.async_copy(table.at[idxv.at[0]], bufs[0], sems[0])
    for c in range(nch):
        cur = c % 2
        if c + 1 < nch:
            nxt = (c + 1) % 2
            handles[nxt] = pltpu.async_copy(
                table.at[idxv.at[c + 1]], bufs[nxt], sems[nxt])
        handles[cur].wait()
        pltpu.sync_copy(bufs[cur], out.at[pl.ds(base + c * ch, ch)])


@functools.cache
def _sc_mesh():
    return plsc.VectorSubcoreMesh(core_axis_name="c", subcore_axis_name="s")


def _sc_gather_cs(emb_city, emb_station, city3, station3):
    out_type = (
        jax.ShapeDtypeStruct((_B, 128), _f32),
        jax.ShapeDtypeStruct((_B, 128), _f32),
    )

    @functools.partial(
        pl.kernel,
        mesh=_sc_mesh(),
        out_type=out_type,
        scratch_types=[
            pltpu.VMEM((_C_NCH, _C_CH), jnp.int32),
            pltpu.VMEM((_C_CH, 128), _f32),
            pltpu.VMEM((_C_CH, 128), _f32),
            pltpu.SemaphoreType.DMA,
            pltpu.SemaphoreType.DMA,
        ],
    )
    def k(ec, es, ci, si, oc, osn, idx_c, bufA, bufB, semA, semB):
        wid = lax.axis_index("s") * 2 + lax.axis_index("c")
        base = wid * _BW
        pltpu.sync_copy(ci.at[wid], idx_c)
        _sc_phase(ec, idx_c, oc, (bufA, bufB), (semA, semB), _C_NCH, _C_CH,
                  base)
        pltpu.sync_copy(si.at[wid], idx_c)
        _sc_phase(es, idx_c, osn, (bufA, bufB), (semA, semB), _C_NCH, _C_CH,
                  base)

    return k(emb_city, emb_station, city3, station3)


_NCK = 4
_CB = _B // _NCK
_D_NCH_K = _CB // _NW // _D_CH


def _sc_gather_d(emb_district, district3k):
    @functools.partial(
        pl.kernel,
        mesh=_sc_mesh(),
        out_type=jax.ShapeDtypeStruct((_CB, 1024), _f32),
        scratch_types=[
            pltpu.VMEM((_D_NCH_K, _D_CH), jnp.int32),
            pltpu.VMEM((_D_CH, 1024), _f32),
            pltpu.VMEM((_D_CH, 1024), _f32),
            pltpu.SemaphoreType.DMA,
            pltpu.SemaphoreType.DMA,
        ],
    )
    def k(ed, di, od, idx_d, bufA, bufB, semA, semB):
        wid = lax.axis_index("s") * 2 + lax.axis_index("c")
        base = wid * (_CB // _NW)
        pltpu.sync_copy(di.at[wid], idx_d)
        _sc_phase(ed, idx_d, od, (bufA, bufB), (semA, semB), _D_NCH_K, _D_CH,
                  base)

    return k(emb_district, district3k)


def _row_spec(n, dtype=None):
    del dtype
    return pl.BlockSpec((_TB, n), lambda i: (i, 0))


def _const_spec(shape):
    nd = len(shape)
    return pl.BlockSpec(shape, lambda i, _nd=nd: (0,) * _nd)


_SEQ = pltpu.CompilerParams(dimension_semantics=("arbitrary",))


_TB1 = 1024
_NT1 = _CB // _TB1


def _p1(gd_k, gc, gs, pref3, emb_pref, wd, wc, ws, wp, b1a, a1, kblk, h1prev):
    def body(gd_ref, gc_ref, gs_ref, pref_ref, ep_ref, wd_ref, wc_ref,
             ws_ref, wp_ref, b_ref, a_ref, *rest):
        h_ref, s_ref, ss_ref = rest[-3:]
        i = pl.program_id(0)
        y = jnp.dot(gd_ref[...].astype(_bf16), wd_ref[...],
                    preferred_element_type=_f32)
        y += jnp.dot(gc_ref[...].astype(_bf16), wc_ref[...],
                     preferred_element_type=_f32)
        y += jnp.dot(gs_ref[...].astype(_bf16), ws_ref[...],
                     preferred_element_type=_f32)
        idx = pref_ref[0, :, :]
        oht = (idx == lax.broadcasted_iota(jnp.int32, (48, _TB1), 0)
               ).astype(_bf16)
        ep = lax.dot_general(oht, ep_ref[...], (((0,), (0,)), ((), ())),
                             preferred_element_type=_f32)
        y += jnp.dot(ep.astype(_bf16), wp_ref[...], preferred_element_type=_f32)
        y += b_ref[...]
        a = a_ref[0, 0]
        h = jnp.where(y >= 0, y, a * y)
        h_ref[...] = h.astype(_bf16)

        @pl.when(i == 0)
        def _():
            s_ref[...] = jnp.zeros_like(s_ref)
            ss_ref[...] = jnp.zeros_like(ss_ref)

        s_ref[...] += jnp.sum(h, axis=0)[None, :]
        ss_ref[...] += jnp.sum(h * h, axis=0)[None, :]

    in_specs = [
        pl.BlockSpec((_TB1, 1024), lambda i: (i, 0)),
        pl.BlockSpec((_TB1, 128), lambda i, k=kblk: (i + k * _NT1, 0)),
        pl.BlockSpec((_TB1, 128), lambda i, k=kblk: (i + k * _NT1, 0)),
        pl.BlockSpec((1, 1, _TB1), lambda i, k=kblk: (i + k * _NT1, 0, 0)),
        _const_spec((48, 10)),
        _const_spec((1024, 1000)), _const_spec((128, 1000)),
        _const_spec((128, 1000)), _const_spec((10, 1000)),
        _const_spec((1, 1000)), _const_spec((1, 1)),
    ]
    args = [gd_k, gc, gs, pref3, emb_pref, wd, wc, ws, wp, b1a, a1]
    aliases = {}
    if h1prev is not None:
        in_specs.append(pl.BlockSpec(memory_space=pl.ANY))
        args.append(h1prev)
        aliases = {11: 0}
    return pl.pallas_call(
        body,
        grid=(_NT1,),
        in_specs=in_specs,
        out_specs=[pl.BlockSpec((_TB1, 1000), lambda i, k=kblk: (i + k * _NT1, 0)),
                   _const_spec((1, 1000)), _const_spec((1, 1000))],
        out_shape=[
            jax.ShapeDtypeStruct((_B, 1000), _bf16),
            jax.ShapeDtypeStruct((1, 1000), _f32),
            jax.ShapeDtypeStruct((1, 1000), _f32),
        ],
        input_output_aliases=aliases,
        compiler_params=_SEQ,
    )(*args)


def _bn_coefs(s, ss, g, be):
    m = s * (1.0 / _B)
    v = ss * (1.0 / _B) - m * m
    scale = g * lax.rsqrt(v + _EPS)
    shift = be - m * scale
    return scale, shift


def _p2(h1, s1, ss1, g1, be1, w1bt, b1b, a2):
    def body(h_ref, s_ref, ss_ref, g_ref, be_ref, w_ref, b_ref, a_ref,
             h2_ref, s2_ref, ss2_ref):
        i = pl.program_id(0)
        scale, shift = _bn_coefs(jnp.sum(s_ref[...], axis=0, keepdims=True),
                                 jnp.sum(ss_ref[...], axis=0, keepdims=True),
                                 g_ref[...], be_ref[...])
        z = h_ref[...].astype(_f32) * scale + shift
        y = jnp.dot(z.astype(_bf16), w_ref[...], preferred_element_type=_f32)
        y += b_ref[...]
        a = a_ref[0, 0]
        h2 = jnp.where(y >= 0, y, a * y)
        h2_ref[...] = h2.astype(_bf16)

        @pl.when(i == 0)
        def _():
            s2_ref[...] = jnp.zeros_like(s2_ref)
            ss2_ref[...] = jnp.zeros_like(ss2_ref)

        s2_ref[...] += jnp.sum(h2, axis=0)[None, :]
        ss2_ref[...] += jnp.sum(h2 * h2, axis=0)[None, :]

    return pl.pallas_call(
        body,
        grid=(_NT,),
        in_specs=[
            _row_spec(1000),
            _const_spec((_NCK, 1000)), _const_spec((_NCK, 1000)),
            _const_spec((1, 1000)), _const_spec((1, 1000)),
            _const_spec((1000, 100)), _const_spec((1, 100)),
            _const_spec((1, 1)),
        ],
        out_specs=[_row_spec(100), _const_spec((1, 100)),
                   _const_spec((1, 100))],
        out_shape=[
            jax.ShapeDtypeStruct((_B, 100), _bf16),
            jax.ShapeDtypeStruct((1, 100), _f32),
            jax.ShapeDtypeStruct((1, 100), _f32),
        ],
        compiler_params=_SEQ,
    )(h1, s1, ss1, g1, be1, w1bt, b1b, a2)


def _p3(h2, x, s2, ss2, g2, be2, w2at_x, w2at_h, b2a):
    def body(h_ref, x_ref, s_ref, ss_ref, g_ref, be_ref, wx_ref, wh_ref,
             b_ref, y3_ref, s3_ref, ss3_ref):
        i = pl.program_id(0)
        scale, shift = _bn_coefs(s_ref[...], ss_ref[...], g_ref[...], be_ref[...])
        z = h_ref[...].astype(_f32) * scale + shift
        y = jnp.dot(x_ref[...].astype(_bf16), wx_ref[...],
                    preferred_element_type=_f32)
        y += jnp.dot(z.astype(_bf16), wh_ref[...], preferred_element_type=_f32)
        y += b_ref[...]
        y3_ref[...] = y.astype(_bf16)

        @pl.when(i == 0)
        def _():
            s3_ref[...] = jnp.zeros_like(s3_ref)
            ss3_ref[...] = jnp.zeros_like(ss3_ref)

        s3_ref[...] += jnp.sum(y, axis=0)[None, :]
        ss3_ref[...] += jnp.sum(y * y, axis=0)[None, :]

    return pl.pallas_call(
        body,
        grid=(_NT,),
        in_specs=[
            _row_spec(100), _row_spec(128),
            _const_spec((1, 100)), _const_spec((1, 100)),
            _const_spec((1, 100)), _const_spec((1, 100)),
            _const_spec((128, 1024)), _const_spec((100, 1024)),
            _const_spec((1, 1024)),
        ],
        out_specs=[_row_spec(1024), _const_spec((1, 1024)),
                   _const_spec((1, 1024))],
        out_shape=[
            jax.ShapeDtypeStruct((_B, 1024), _bf16),
            jax.ShapeDtypeStruct((1, 1024), _f32),
            jax.ShapeDtypeStruct((1, 1024), _f32),
        ],
        compiler_params=_SEQ,
    )(h2, x, s2, ss2, g2, be2, w2at_x, w2at_h, b2a)


def _p4(y3, s3, ss3, g3, be3, w2bt, b2b):
    def body(y3_ref, s_ref, ss_ref, g_ref, be_ref, w_ref, b_ref,
             y4_ref, s4_ref, ss4_ref):
        i = pl.program_id(0)
        scale, shift = _bn_coefs(s_ref[...], ss_ref[...], g_ref[...], be_ref[...])
        r = jnp.maximum(y3_ref[...].astype(_f32) * scale + shift, 0.0)
        y = jnp.dot(r.astype(_bf16), w_ref[...], preferred_element_type=_f32)
        y += b_ref[...]
        y4_ref[...] = y.astype(_bf16)

        @pl.when(i == 0)
        def _():
            s4_ref[...] = jnp.zeros_like(s4_ref)
            ss4_ref[...] = jnp.zeros_like(ss4_ref)

        s4_ref[...] += jnp.sum(y, axis=0)[None, :]
        ss4_ref[...] += jnp.sum(y * y, axis=0)[None, :]

    return pl.pallas_call(
        body,
        grid=(_NT,),
        in_specs=[
            _row_spec(1024),
            _const_spec((1, 1024)), _const_spec((1, 1024)),
            _const_spec((1, 1024)), _const_spec((1, 1024)),
            _const_spec((1024, 512)), _const_spec((1, 512)),
        ],
        out_specs=[_row_spec(512), _const_spec((1, 512)),
                   _const_spec((1, 512))],
        out_shape=[
            jax.ShapeDtypeStruct((_B, 512), _bf16),
            jax.ShapeDtypeStruct((1, 512), _f32),
            jax.ShapeDtypeStruct((1, 512), _f32),
        ],
        compiler_params=_SEQ,
    )(y3, s3, ss3, g3, be3, w2bt, b2b)


def _p5(y4, s4, ss4, g4, be4, w2c, b2c):
    def body(y4_ref, s_ref, ss_ref, g_ref, be_ref, w_ref, b_ref, o_ref):
        scale, shift = _bn_coefs(s_ref[...], ss_ref[...], g_ref[...], be_ref[...])
        r = jnp.maximum(y4_ref[...].astype(_f32) * scale + shift, 0.0)
        o = jnp.sum(r * w_ref[...], axis=1, keepdims=True) + b_ref[0, 0]
        o_ref[...] = o

    return pl.pallas_call(
        body,
        grid=(_NT,),
        in_specs=[
            _row_spec(512),
            _const_spec((1, 512)), _const_spec((1, 512)),
            _const_spec((1, 512)), _const_spec((1, 512)),
            _const_spec((1, 512)), _const_spec((1, 1)),
        ],
        out_specs=[_row_spec(1)],
        out_shape=[jax.ShapeDtypeStruct((_B, 1), _f32)],
        compiler_params=_SEQ,
    )(y4, s4, ss4, g4, be4, w2c, b2c)[0]


def kernel(x, pref, city, district, station,
           emb_pref, emb_city, emb_district, emb_station,
           W1a, b1a, a1, g1, be1,
           W1b, b1b, a2, g2, be2,
           W2a, b2a, g3, be3,
           W2b, b2b, g4, be4,
           W2c, b2c):
    city3 = city.reshape(_NW, _C_NCH, _C_CH)
    station3 = station.reshape(_NW, _C_NCH, _C_CH)
    district4 = district.reshape(_NCK, _NW, _D_NCH_K, _D_CH)

    gc, gs = _sc_gather_cs(_pad_cols(emb_city, 128),
                           _pad_cols(emb_station, 128),
                           city3, station3)
    ed_p = _transpose_pad(emb_district.T, 1024)
    gds = [_sc_gather_d(ed_p, district4[k]) for k in range(_NCK)]

    wp = W1a[:, 0:10].T.astype(_bf16)
    wc = jnp.pad(W1a[:, 10:110].T, ((0, 28), (0, 0))).astype(_bf16)
    wd = jnp.pad(W1a[:, 110:1110].T, ((0, 24), (0, 0))).astype(_bf16)
    ws = jnp.pad(W1a[:, 1110:1210].T, ((0, 28), (0, 0))).astype(_bf16)
    pref3 = pref.reshape(_B // _TB1, 1, _TB1)
    ep16 = emb_pref.astype(_bf16)
    b1a2 = b1a.reshape(1, 1000)
    a12 = a1.reshape(1, 1)

    h1 = None
    sparts, ssparts = [], []
    for k in range(_NCK):
        h1, sk, ssk = _p1(gds[k], gc, gs, pref3, ep16, wd, wc, ws, wp,
                          b1a2, a12, k, h1)
        sparts.append(sk)
        ssparts.append(ssk)
    s1 = jnp.concatenate(sparts, axis=0)
    ss1 = jnp.concatenate(ssparts, axis=0)

    h2, s2, ss2 = _p2(h1, s1, ss1, g1.reshape(1, 1000), be1.reshape(1, 1000),
                      W1b.T.astype(_bf16), b1b.reshape(1, 100),
                      a2.reshape(1, 1))

    w2at = W2a.T.astype(_bf16)
    y3, s3, ss3 = _p3(h2, x, s2, ss2, g2.reshape(1, 100), be2.reshape(1, 100),
                      w2at[0:128], w2at[128:228], b2a.reshape(1, 1024))

    y4, s4, ss4 = _p4(y3, s3, ss3, g3.reshape(1, 1024), be3.reshape(1, 1024),
                      W2b.T.astype(_bf16), b2b.reshape(1, 512))

    return _p5(y4, s4, ss4, g4.reshape(1, 512), be4.reshape(1, 512),
               W2c.reshape(1, 512), b2c.reshape(1, 1))


# speedup vs baseline: 1.0501x; 1.0501x over previous
import functools

import jax
import jax.numpy as jnp
from jax import lax
from jax.experimental import pallas as pl
from jax.experimental.pallas import tpu as pltpu
from jax.experimental.pallas import tpu_sc as plsc

_B = 16384
_TB = 2048
_NT = _B // _TB
_NW = 32
_BW = _B // _NW
_EPS = 1e-5

_D_CH, _D_NCH = 32, 16
_C_CH, _C_NCH = 128, 4

_f32 = jnp.float32
_bf16 = jnp.bfloat16


def _pad_cols(arr, cp):
    r, c = arr.shape
    rb = min(r, 2048)
    grid = (r + rb - 1) // rb

    def body(in_ref, out_ref):
        out_ref[:, 0:c] = in_ref[...]
        out_ref[:, c:cp] = jnp.zeros((rb, cp - c), arr.dtype)

    return pl.pallas_call(
        body,
        grid=(grid,),
        in_specs=[pl.BlockSpec((rb, c), lambda i: (i, 0))],
        out_specs=pl.BlockSpec((rb, cp), lambda i: (i, 0)),
        out_shape=jax.ShapeDtypeStruct((r, cp), arr.dtype),
        compiler_params=_SEQ,
    )(arr)


def _transpose_pad(arr_t, cp):
    c, r = arr_t.shape
    rb = 512
    grid = (r + rb - 1) // rb

    def body(in_ref, out_ref):
        out_ref[:, 0:c] = in_ref[...].T
        out_ref[:, c:cp] = jnp.zeros((rb, cp - c), arr_t.dtype)

    return pl.pallas_call(
        body,
        grid=(grid,),
        in_specs=[pl.BlockSpec((c, rb), lambda i: (0, i))],
        out_specs=pl.BlockSpec((rb, cp), lambda i: (i, 0)),
        out_shape=jax.ShapeDtypeStruct((r, cp), arr_t.dtype),
        compiler_params=_SEQ,
    )(arr_t)


def _sc_phase(table, idxv, out, bufs, sems, nch, ch, base):
    handles = [None, None]
    handles[0] = pltpu.async_copy(table.at[idxv.at[0]], bufs[0], sems[0])
    for c in range(nch):
        cur = c % 2
        if c + 1 < nch:
            nxt = (c + 1) % 2
            handles[nxt] = pltpu.async_copy(
                table.at[idxv.at[c + 1]], bufs[nxt], sems[nxt])
        handles[cur].wait()
        pltpu.sync_copy(bufs[cur], out.at[pl.ds(base + c * ch, ch)])


@functools.cache
def _sc_mesh():
    return plsc.VectorSubcoreMesh(core_axis_name="c", subcore_axis_name="s")


def _sc_gather_cs(emb_city, emb_station, city3, station3):
    out_type = (
        jax.ShapeDtypeStruct((_B, 128), _f32),
        jax.ShapeDtypeStruct((_B, 128), _f32),
    )

    @functools.partial(
        pl.kernel,
        mesh=_sc_mesh(),
        out_type=out_type,
        scratch_types=[
            pltpu.VMEM((_C_NCH, _C_CH), jnp.int32),
            pltpu.VMEM((_C_CH, 128), _f32),
            pltpu.VMEM((_C_CH, 128), _f32),
            pltpu.SemaphoreType.DMA,
            pltpu.SemaphoreType.DMA,
        ],
    )
    def k(ec, es, ci, si, oc, osn, idx_c, bufA, bufB, semA, semB):
        wid = lax.axis_index("s") * 2 + lax.axis_index("c")
        base = wid * _BW
        pltpu.sync_copy(ci.at[wid], idx_c)
        _sc_phase(ec, idx_c, oc, (bufA, bufB), (semA, semB), _C_NCH, _C_CH,
                  base)
        pltpu.sync_copy(si.at[wid], idx_c)
        _sc_phase(es, idx_c, osn, (bufA, bufB), (semA, semB), _C_NCH, _C_CH,
                  base)

    return k(emb_city, emb_station, city3, station3)


_NCK = 4
_CB = _B // _NCK
_D_NCH_K = _CB // _NW // _D_CH


def _sc_gather_d(emb_district, district3k):
    @functools.partial(
        pl.kernel,
        mesh=_sc_mesh(),
        out_type=jax.ShapeDtypeStruct((_CB, 1024), _f32),
        scratch_types=[
            pltpu.VMEM((_D_NCH_K, _D_CH), jnp.int32),
            pltpu.VMEM((_D_CH, 1024), _f32),
            pltpu.VMEM((_D_CH, 1024), _f32),
            pltpu.SemaphoreType.DMA,
            pltpu.SemaphoreType.DMA,
        ],
    )
    def k(ed, di, od, idx_d, bufA, bufB, semA, semB):
        wid = lax.axis_index("s") * 2 + lax.axis_index("c")
        base = wid * (_CB // _NW)
        pltpu.sync_copy(di.at[wid], idx_d)
        _sc_phase(ed, idx_d, od, (bufA, bufB), (semA, semB), _D_NCH_K, _D_CH,
                  base)

    return k(emb_district, district3k)


def _row_spec(n, dtype=None):
    del dtype
    return pl.BlockSpec((_TB, n), lambda i: (i, 0))


def _const_spec(shape):
    nd = len(shape)
    return pl.BlockSpec(shape, lambda i, _nd=nd: (0,) * _nd)


_SEQ = pltpu.CompilerParams(dimension_semantics=("arbitrary",))


_TB1 = 1024
_NT1 = _CB // _TB1


def _p1(gd_k, gc, gs, pref3, emb_pref, wd, wc, ws, wp, b1a, a1, kblk, h1prev):
    def body(gd_ref, gc_ref, gs_ref, pref_ref, ep_ref, wd_ref, wc_ref,
             ws_ref, wp_ref, b_ref, a_ref, *rest):
        h_ref, s_ref, ss_ref = rest[-3:]
        i = pl.program_id(0)
        y = jnp.dot(gd_ref[...].astype(_bf16), wd_ref[...],
                    preferred_element_type=_f32)
        y += jnp.dot(gc_ref[...].astype(_bf16), wc_ref[...],
                     preferred_element_type=_f32)
        y += jnp.dot(gs_ref[...].astype(_bf16), ws_ref[...],
                     preferred_element_type=_f32)
        idx = pref_ref[0, :, :]
        oht = (idx == lax.broadcasted_iota(jnp.int32, (48, _TB1), 0)
               ).astype(_bf16)
        ep = lax.dot_general(oht, ep_ref[...], (((0,), (0,)), ((), ())),
                             preferred_element_type=_f32)
        y += jnp.dot(ep.astype(_bf16), wp_ref[...], preferred_element_type=_f32)
        y += b_ref[...]
        a = a_ref[0, 0]
        h = jnp.where(y >= 0, y, a * y)
        h_ref[...] = h.astype(_bf16)

        @pl.when(i == 0)
        def _():
            s_ref[...] = jnp.zeros_like(s_ref)
            ss_ref[...] = jnp.zeros_like(ss_ref)

        s_ref[...] += jnp.sum(h, axis=0)[None, :]
        ss_ref[...] += jnp.sum(h * h, axis=0)[None, :]

    in_specs = [
        pl.BlockSpec((_TB1, 1024), lambda i: (i, 0)),
        pl.BlockSpec((_TB1, 128), lambda i, k=kblk: (i + k * _NT1, 0)),
        pl.BlockSpec((_TB1, 128), lambda i, k=kblk: (i + k * _NT1, 0)),
        pl.BlockSpec((1, 1, _TB1), lambda i, k=kblk: (i + k * _NT1, 0, 0)),
        _const_spec((48, 10)),
        _const_spec((1024, 1000)), _const_spec((128, 1000)),
        _const_spec((128, 1000)), _const_spec((10, 1000)),
        _const_spec((1, 1000)), _const_spec((1, 1)),
    ]
    args = [gd_k, gc, gs, pref3, emb_pref, wd, wc, ws, wp, b1a, a1]
    aliases = {}
    if h1prev is not None:
        in_specs.append(pl.BlockSpec(memory_space=pl.ANY))
        args.append(h1prev)
        aliases = {11: 0}
    return pl.pallas_call(
        body,
        grid=(_NT1,),
        in_specs=in_specs,
        out_specs=[pl.BlockSpec((_TB1, 1000), lambda i, k=kblk: (i + k * _NT1, 0)),
                   _const_spec((1, 1000)), _const_spec((1, 1000))],
        out_shape=[
            jax.ShapeDtypeStruct((_B, 1000), _bf16),
            jax.ShapeDtypeStruct((1, 1000), _f32),
            jax.ShapeDtypeStruct((1, 1000), _f32),
        ],
        input_output_aliases=aliases,
        compiler_params=_SEQ,
    )(*args)


def _bn_coefs(s, ss, g, be):
    m = s * (1.0 / _B)
    v = ss * (1.0 / _B) - m * m
    scale = g * lax.rsqrt(v + _EPS)
    shift = be - m * scale
    return scale, shift


def _p2(h1, s1, ss1, g1, be1, w1bt, b1b, a2):
    def body(h_ref, s_ref, ss_ref, g_ref, be_ref, w_ref, b_ref, a_ref,
             h2_ref, s2_ref, ss2_ref):
        i = pl.program_id(0)
        scale, shift = _bn_coefs(jnp.sum(s_ref[...], axis=0, keepdims=True),
                                 jnp.sum(ss_ref[...], axis=0, keepdims=True),
                                 g_ref[...], be_ref[...])
        z = h_ref[...].astype(_f32) * scale + shift
        y = jnp.dot(z.astype(_bf16), w_ref[...], preferred_element_type=_f32)
        y += b_ref[...]
        a = a_ref[0, 0]
        h2 = jnp.where(y >= 0, y, a * y)
        h2_ref[...] = h2.astype(_bf16)

        @pl.when(i == 0)
        def _():
            s2_ref[...] = jnp.zeros_like(s2_ref)
            ss2_ref[...] = jnp.zeros_like(ss2_ref)

        s2_ref[...] += jnp.sum(h2, axis=0)[None, :]
        ss2_ref[...] += jnp.sum(h2 * h2, axis=0)[None, :]

    return pl.pallas_call(
        body,
        grid=(_NT,),
        in_specs=[
            _row_spec(1000),
            _const_spec((_NCK, 1000)), _const_spec((_NCK, 1000)),
            _const_spec((1, 1000)), _const_spec((1, 1000)),
            _const_spec((1000, 100)), _const_spec((1, 100)),
            _const_spec((1, 1)),
        ],
        out_specs=[_row_spec(100), _const_spec((1, 100)),
                   _const_spec((1, 100))],
        out_shape=[
            jax.ShapeDtypeStruct((_B, 100), _bf16),
            jax.ShapeDtypeStruct((1, 100), _f32),
            jax.ShapeDtypeStruct((1, 100), _f32),
        ],
        compiler_params=_SEQ,
    )(h1, s1, ss1, g1, be1, w1bt, b1b, a2)


def _p3(h2, x, s2, ss2, g2, be2, w2at_x, w2at_h, b2a):
    def body(h_ref, x_ref, s_ref, ss_ref, g_ref, be_ref, wx_ref, wh_ref,
             b_ref, y3_ref, s3_ref, ss3_ref):
        i = pl.program_id(0)
        scale, shift = _bn_coefs(s_ref[...], ss_ref[...], g_ref[...], be_ref[...])
        z = h_ref[...].astype(_f32) * scale + shift
        y = jnp.dot(x_ref[...].astype(_bf16), wx_ref[...],
                    preferred_element_type=_f32)
        y += jnp.dot(z.astype(_bf16), wh_ref[...], preferred_element_type=_f32)
        y += b_ref[...]
        y3_ref[...] = y.astype(_bf16)

        @pl.when(i == 0)
        def _():
            s3_ref[...] = jnp.zeros_like(s3_ref)
            ss3_ref[...] = jnp.zeros_like(ss3_ref)

        s3_ref[...] += jnp.sum(y, axis=0)[None, :]
        ss3_ref[...] += jnp.sum(y * y, axis=0)[None, :]

    return pl.pallas_call(
        body,
        grid=(_NT,),
        in_specs=[
            _row_spec(100), _row_spec(128),
            _const_spec((1, 100)), _const_spec((1, 100)),
            _const_spec((1, 100)), _const_spec((1, 100)),
            _const_spec((128, 1024)), _const_spec((100, 1024)),
            _const_spec((1, 1024)),
        ],
        out_specs=[_row_spec(1024), _const_spec((1, 1024)),
                   _const_spec((1, 1024))],
        out_shape=[
            jax.ShapeDtypeStruct((_B, 1024), _bf16),
            jax.ShapeDtypeStruct((1, 1024), _f32),
            jax.ShapeDtypeStruct((1, 1024), _f32),
        ],
        compiler_params=_SEQ,
    )(h2, x, s2, ss2, g2, be2, w2at_x, w2at_h, b2a)


def _p4(y3, s3, ss3, g3, be3, w2bt, b2b):
    def body(y3_ref, s_ref, ss_ref, g_ref, be_ref, w_ref, b_ref,
             y4_ref, s4_ref, ss4_ref):
        i = pl.program_id(0)
        scale, shift = _bn_coefs(s_ref[...], ss_ref[...], g_ref[...], be_ref[...])
        r = jnp.maximum(y3_ref[...].astype(_f32) * scale + shift, 0.0)
        y = jnp.dot(r.astype(_bf16), w_ref[...], preferred_element_type=_f32)
        y += b_ref[...]
        y4_ref[...] = y.astype(_bf16)

        @pl.when(i == 0)
        def _():
            s4_ref[...] = jnp.zeros_like(s4_ref)
            ss4_ref[...] = jnp.zeros_like(ss4_ref)

        s4_ref[...] += jnp.sum(y, axis=0)[None, :]
        ss4_ref[...] += jnp.sum(y * y, axis=0)[None, :]

    return pl.pallas_call(
        body,
        grid=(_NT,),
        in_specs=[
            _row_spec(1024),
            _const_spec((1, 1024)), _const_spec((1, 1024)),
            _const_spec((1, 1024)), _const_spec((1, 1024)),
            _const_spec((1024, 512)), _const_spec((1, 512)),
        ],
        out_specs=[_row_spec(512), _const_spec((1, 512)),
                   _const_spec((1, 512))],
        out_shape=[
            jax.ShapeDtypeStruct((_B, 512), _bf16),
            jax.ShapeDtypeStruct((1, 512), _f32),
            jax.ShapeDtypeStruct((1, 512), _f32),
        ],
        compiler_params=_SEQ,
    )(y3, s3, ss3, g3, be3, w2bt, b2b)


def _p5(y4, s4, ss4, g4, be4, w2c, b2c):
    def body(y4_ref, s_ref, ss_ref, g_ref, be_ref, w_ref, b_ref, o_ref):
        scale, shift = _bn_coefs(s_ref[...], ss_ref[...], g_ref[...], be_ref[...])
        r = jnp.maximum(y4_ref[...].astype(_f32) * scale + shift, 0.0)
        o = jnp.sum(r * w_ref[...], axis=1, keepdims=True) + b_ref[0, 0]
        o_ref[...] = o

    return pl.pallas_call(
        body,
        grid=(_NT,),
        in_specs=[
            _row_spec(512),
            _const_spec((1, 512)), _const_spec((1, 512)),
            _const_spec((1, 512)), _const_spec((1, 512)),
            _const_spec((1, 512)), _const_spec((1, 1)),
        ],
        out_specs=[_row_spec(1)],
        out_shape=[jax.ShapeDtypeStruct((_B, 1), _f32)],
        compiler_params=_SEQ,
    )(y4, s4, ss4, g4, be4, w2c, b2c)[0]


def kernel(x, pref, city, district, station,
           emb_pref, emb_city, emb_district, emb_station,
           W1a, b1a, a1, g1, be1,
           W1b, b1b, a2, g2, be2,
           W2a, b2a, g3, be3,
           W2b, b2b, g4, be4,
           W2c, b2c):
    city3 = city.reshape(_NW, _C_NCH, _C_CH)
    station3 = station.reshape(_NW, _C_NCH, _C_CH)
    district4 = district.reshape(_NCK, _NW, _D_NCH_K, _D_CH)

    gc, gs = _sc_gather_cs(_pad_cols(emb_city, 128),
                           _pad_cols(emb_station, 128),
                           city3, station3)
    ed_p = _transpose_pad(emb_district.T, 1024)
    gds = [_sc_gather_d(ed_p, district4[k]) for k in range(_NCK)]

    wp = W1a[:, 0:10].T.astype(_bf16)
    wc = jnp.pad(W1a[:, 10:110].T, ((0, 28), (0, 0))).astype(_bf16)
    wd = jnp.pad(W1a[:, 110:1110].T, ((0, 24), (0, 0))).astype(_bf16)
    ws = jnp.pad(W1a[:, 1110:1210].T, ((0, 28), (0, 0))).astype(_bf16)
    pref3 = pref.reshape(_B // _TB1, 1, _TB1)
    ep16 = emb_pref.astype(_bf16)
    b1a2 = b1a.reshape(1, 1000)
    a12 = a1.reshape(1, 1)

    h1 = None
    sparts, ssparts = [], []
    for k in range(_NCK):
        h1, sk, ssk = _p1(gds[k], gc, gs, pref3, ep16, wd, wc, ws, wp,
                          b1a2, a12, k, h1)
        sparts.append(sk)
        ssparts.append(ssk)
    s1 = jnp.concatenate(sparts, axis=0)
    ss1 = jnp.concatenate(ssparts, axis=0)

    h2, s2, ss2 = _p2(h1, s1, ss1, g1.reshape(1, 1000), be1.reshape(1, 1000),
                      W1b.T.astype(_bf16), b1b.reshape(1, 100),
                      a2.reshape(1, 1))

    w2at = W2a.T.astype(_bf16)
    y3, s3, ss3 = _p3(h2, x, s2, ss2, g2.reshape(1, 100), be2.reshape(1, 100),
                      w2at[0:128], w2at[128:228], b2a.reshape(1, 1024))

    y4, s4, ss4 = _p4(y3, s3, ss3, g3.reshape(1, 1024), be3.reshape(1, 1024),
                      W2b.T.astype(_bf16), b2b.reshape(1, 512))

    return _p5(y4, s4, ss4, g4.reshape(1, 512), be4.reshape(1, 512),
               W2c.reshape(1, 512), b2c.reshape(1, 1))


# speedup vs baseline: 1.0815x; 1.0299x over previous
import functools

import jax
import jax.numpy as jnp
from jax import lax
from jax.experimental import pallas as pl
from jax.experimental.pallas import tpu as pltpu
from jax.experimental.pallas import tpu_sc as plsc

_B = 16384
_TB = 2048
_NT = _B // _TB
_NW = 32
_BW = _B // _NW
_EPS = 1e-5

_D_CH, _D_NCH = 32, 16
_C_CH, _C_NCH = 128, 4

_f32 = jnp.float32
_bf16 = jnp.bfloat16


def _pad_cols(arr, cp):
    r, c = arr.shape
    rb = min(r, 2048)
    grid = (r + rb - 1) // rb

    def body(in_ref, out_ref):
        out_ref[:, 0:c] = in_ref[...]
        out_ref[:, c:cp] = jnp.zeros((rb, cp - c), arr.dtype)

    return pl.pallas_call(
        body,
        grid=(grid,),
        in_specs=[pl.BlockSpec((rb, c), lambda i: (i, 0))],
        out_specs=pl.BlockSpec((rb, cp), lambda i: (i, 0)),
        out_shape=jax.ShapeDtypeStruct((r, cp), arr.dtype),
        compiler_params=_SEQ,
    )(arr)


def _transpose_pad(arr_t, cp):
    c, r = arr_t.shape
    rb = 512
    grid = (r + rb - 1) // rb

    def body(in_ref, out_ref):
        out_ref[:, 0:c] = in_ref[...].T
        out_ref[:, c:cp] = jnp.zeros((rb, cp - c), arr_t.dtype)

    return pl.pallas_call(
        body,
        grid=(grid,),
        in_specs=[pl.BlockSpec((c, rb), lambda i: (0, i))],
        out_specs=pl.BlockSpec((rb, cp), lambda i: (i, 0)),
        out_shape=jax.ShapeDtypeStruct((r, cp), arr_t.dtype),
        compiler_params=_SEQ,
    )(arr_t)


def _sc_phase(table, idxv, out, bufs, sems, nch, ch, base):
    handles = [None, None]
    handles[0] = pltpu.async_copy(table.at[idxv.at[0]], bufs[0], sems[0])
    for c in range(nch):
        cur = c % 2
        if c + 1 < nch:
            nxt = (c + 1) % 2
            handles[nxt] = pltpu.async_copy(
                table.at[idxv.at[c + 1]], bufs[nxt], sems[nxt])
        handles[cur].wait()
        pltpu.sync_copy(bufs[cur], out.at[pl.ds(base + c * ch, ch)])


@functools.cache
def _sc_mesh():
    return plsc.VectorSubcoreMesh(core_axis_name="c", subcore_axis_name="s")


def _sc_gather_cs(emb_city, emb_station, city3, station3):
    out_type = (
        jax.ShapeDtypeStruct((_B, 128), _f32),
        jax.ShapeDtypeStruct((_B, 128), _f32),
    )

    @functools.partial(
        pl.kernel,
        mesh=_sc_mesh(),
        out_type=out_type,
        scratch_types=[
            pltpu.VMEM((_C_NCH, _C_CH), jnp.int32),
            pltpu.VMEM((_C_CH, 128), _f32),
            pltpu.VMEM((_C_CH, 128), _f32),
            pltpu.SemaphoreType.DMA,
            pltpu.SemaphoreType.DMA,
        ],
    )
    def k(ec, es, ci, si, oc, osn, idx_c, bufA, bufB, semA, semB):
        wid = lax.axis_index("s") * 2 + lax.axis_index("c")
        base = wid * _BW
        pltpu.sync_copy(ci.at[wid], idx_c)
        _sc_phase(ec, idx_c, oc, (bufA, bufB), (semA, semB), _C_NCH, _C_CH,
                  base)
        pltpu.sync_copy(si.at[wid], idx_c)
        _sc_phase(es, idx_c, osn, (bufA, bufB), (semA, semB), _C_NCH, _C_CH,
                  base)

    return k(emb_city, emb_station, city3, station3)


_NCK = 4
_CB = _B // _NCK
_D_NCH_K = _CB // _NW // _D_CH


def _sc_gather_d(emb_district, district3k):
    @functools.partial(
        pl.kernel,
        mesh=_sc_mesh(),
        out_type=jax.ShapeDtypeStruct((_CB, 1024), _f32),
        scratch_types=[
            pltpu.VMEM((_D_NCH_K, _D_CH), jnp.int32),
            pltpu.VMEM((_D_CH, 1024), _f32),
            pltpu.VMEM((_D_CH, 1024), _f32),
            pltpu.SemaphoreType.DMA,
            pltpu.SemaphoreType.DMA,
        ],
    )
    def k(ed, di, od, idx_d, bufA, bufB, semA, semB):
        wid = lax.axis_index("s") * 2 + lax.axis_index("c")
        base = wid * (_CB // _NW)
        pltpu.sync_copy(di.at[wid], idx_d)
        _sc_phase(ed, idx_d, od, (bufA, bufB), (semA, semB), _D_NCH_K, _D_CH,
                  base)

    return k(emb_district, district3k)


def _row_spec(n, dtype=None):
    del dtype
    return pl.BlockSpec((_TB, n), lambda i: (i, 0))


def _const_spec(shape):
    nd = len(shape)
    return pl.BlockSpec(shape, lambda i, _nd=nd: (0,) * _nd)


_SEQ = pltpu.CompilerParams(dimension_semantics=("arbitrary",))


_TB1 = 1024
_NT1 = _CB // _TB1


def _p1(gd_k, gc, gs, pref3, emb_pref, wd, wc, ws, wp, b1a, a1, kblk, h1prev):
    def body(gd_ref, gc_ref, gs_ref, pref_ref, ep_ref, wd_ref, wc_ref,
             ws_ref, wp_ref, b_ref, a_ref, *rest):
        h_ref, s_ref, ss_ref = rest[-3:]
        i = pl.program_id(0)
        y = jnp.dot(gd_ref[...].astype(_bf16), wd_ref[...],
                    preferred_element_type=_f32)
        y += jnp.dot(gc_ref[...].astype(_bf16), wc_ref[...],
                     preferred_element_type=_f32)
        y += jnp.dot(gs_ref[...].astype(_bf16), ws_ref[...],
                     preferred_element_type=_f32)
        idx = pref_ref[0, :, :]
        oht = (idx == lax.broadcasted_iota(jnp.int32, (48, _TB1), 0)
               ).astype(_bf16)
        ep = lax.dot_general(oht, ep_ref[...], (((0,), (0,)), ((), ())),
                             preferred_element_type=_f32)
        y += jnp.dot(ep.astype(_bf16), wp_ref[...], preferred_element_type=_f32)
        y += b_ref[...]
        a = a_ref[0, 0]
        h = jnp.where(y >= 0, y, a * y)
        h_ref[...] = h.astype(_bf16)

        @pl.when(i == 0)
        def _():
            s_ref[...] = jnp.zeros_like(s_ref)
            ss_ref[...] = jnp.zeros_like(ss_ref)

        s_ref[...] += jnp.sum(h, axis=0)[None, :]
        ss_ref[...] += jnp.sum(h * h, axis=0)[None, :]

    in_specs = [
        pl.BlockSpec((_TB1, 1024), lambda i: (i, 0)),
        pl.BlockSpec((_TB1, 128), lambda i, k=kblk: (i + k * _NT1, 0)),
        pl.BlockSpec((_TB1, 128), lambda i, k=kblk: (i + k * _NT1, 0)),
        pl.BlockSpec((1, 1, _TB1), lambda i, k=kblk: (i + k * _NT1, 0, 0)),
        _const_spec((48, 10)),
        _const_spec((1024, 1000)), _const_spec((128, 1000)),
        _const_spec((128, 1000)), _const_spec((10, 1000)),
        _const_spec((1, 1000)), _const_spec((1, 1)),
    ]
    args = [gd_k, gc, gs, pref3, emb_pref, wd, wc, ws, wp, b1a, a1]
    aliases = {}
    if h1prev is not None:
        in_specs.append(pl.BlockSpec(memory_space=pl.ANY))
        args.append(h1prev)
        aliases = {11: 0}
    return pl.pallas_call(
        body,
        grid=(_NT1,),
        in_specs=in_specs,
        out_specs=[pl.BlockSpec((_TB1, 1000), lambda i, k=kblk: (i + k * _NT1, 0)),
                   _const_spec((1, 1000)), _const_spec((1, 1000))],
        out_shape=[
            jax.ShapeDtypeStruct((_B, 1000), _bf16),
            jax.ShapeDtypeStruct((1, 1000), _f32),
            jax.ShapeDtypeStruct((1, 1000), _f32),
        ],
        input_output_aliases=aliases,
        compiler_params=_SEQ,
    )(*args)


def _bn_coefs(s, ss, g, be):
    m = s * (1.0 / _B)
    v = ss * (1.0 / _B) - m * m
    scale = g * lax.rsqrt(v + _EPS)
    shift = be - m * scale
    return scale, shift


def _p2(h1, s1, ss1, x, g1, be1, w1bt, b1b, a2):
    def body(h_ref, s_ref, ss_ref, x_ref, g_ref, be_ref, w_ref, b_ref, a_ref,
             h2_ref, s2_ref, ss2_ref, m_ref, sv_ref):
        i = pl.program_id(0)
        scale, shift = _bn_coefs(jnp.sum(s_ref[...], axis=0, keepdims=True),
                                 jnp.sum(ss_ref[...], axis=0, keepdims=True),
                                 g_ref[...], be_ref[...])
        z = h_ref[...].astype(_f32) * scale + shift
        y = jnp.dot(z.astype(_bf16), w_ref[...], preferred_element_type=_f32)
        y += b_ref[...]
        a = a_ref[0, 0]
        h2 = jnp.where(y >= 0, y, a * y)
        h2_ref[...] = h2.astype(_bf16)

        @pl.when(i == 0)
        def _():
            s2_ref[...] = jnp.zeros_like(s2_ref)
            ss2_ref[...] = jnp.zeros_like(ss2_ref)
            m_ref[...] = jnp.zeros_like(m_ref)
            sv_ref[...] = jnp.zeros_like(sv_ref)

        s2_ref[...] += jnp.sum(h2, axis=0)[None, :]
        ss2_ref[...] += jnp.sum(h2 * h2, axis=0)[None, :]
        v = jnp.concatenate([x_ref[...], h2], axis=1)
        m_ref[...] += lax.dot_general(v, v, (((0,), (0,)), ((), ())),
                                      preferred_element_type=_f32)
        sv_ref[...] += jnp.sum(v, axis=0)[None, :]

    return pl.pallas_call(
        body,
        grid=(_NT,),
        in_specs=[
            _row_spec(1000),
            _const_spec((_NCK, 1000)), _const_spec((_NCK, 1000)),
            _row_spec(128),
            _const_spec((1, 1000)), _const_spec((1, 1000)),
            _const_spec((1000, 100)), _const_spec((1, 100)),
            _const_spec((1, 1)),
        ],
        out_specs=[_row_spec(100), _const_spec((1, 100)),
                   _const_spec((1, 100)), _const_spec((228, 228)),
                   _const_spec((1, 228))],
        out_shape=[
            jax.ShapeDtypeStruct((_B, 100), _bf16),
            jax.ShapeDtypeStruct((1, 100), _f32),
            jax.ShapeDtypeStruct((1, 100), _f32),
            jax.ShapeDtypeStruct((228, 228), _f32),
            jax.ShapeDtypeStruct((1, 228), _f32),
        ],
        compiler_params=_SEQ,
    )(h1, s1, ss1, x, g1, be1, w1bt, b1b, a2)


def _p34(h2, x, s2, ss2, m2, sv2, g2, be2, w2at_x, w2at_h, w2at_f, b2a,
         g3, be3, w2bt, b2b):
    def body(h_ref, x_ref, s_ref, ss_ref, m_ref, sv_ref, g2_ref, be2_ref,
             wx_ref, wh_ref, wf_ref, ba_ref, g3_ref, be3_ref, wb_ref, bb_ref,
             y4_ref, s4_ref, ss4_ref, sc3_ref, sh3_ref):
        i = pl.program_id(0)
        scale2, shift2 = _bn_coefs(s_ref[...], ss_ref[...],
                                   g2_ref[...], be2_ref[...])

        @pl.when(i == 0)
        def _():
            mu = sv_ref[...] * (1.0 / _B)
            cov = m_ref[...] * (1.0 / _B) - jnp.transpose(mu) * mu
            dvec = jnp.concatenate(
                [jnp.ones((1, 128), _f32), scale2], axis=1)
            covd = cov * dvec * jnp.transpose(dvec)
            wf = wf_ref[...]
            t = jnp.dot(covd, wf, preferred_element_type=_f32)
            v3 = jnp.sum(wf * t, axis=0, keepdims=True)
            mu_u = jnp.concatenate([mu[:, 0:128], be2_ref[...]], axis=1)
            m3 = jnp.dot(mu_u, wf, preferred_element_type=_f32) + ba_ref[...]
            sc3 = g3_ref[...] * lax.rsqrt(v3 + _EPS)
            sc3_ref[...] = sc3
            sh3_ref[...] = be3_ref[...] - m3 * sc3
            s4_ref[...] = jnp.zeros_like(s4_ref)
            ss4_ref[...] = jnp.zeros_like(ss4_ref)

        z = h_ref[...].astype(_f32) * scale2 + shift2
        y3 = jnp.dot(x_ref[...].astype(_bf16), wx_ref[...],
                     preferred_element_type=_f32)
        y3 += jnp.dot(z.astype(_bf16), wh_ref[...], preferred_element_type=_f32)
        y3 += ba_ref[...]
        r = jnp.maximum(y3 * sc3_ref[...] + sh3_ref[...], 0.0)
        y4 = jnp.dot(r.astype(_bf16), wb_ref[...], preferred_element_type=_f32)
        y4 += bb_ref[...]
        y4_ref[...] = y4.astype(_bf16)
        s4_ref[...] += jnp.sum(y4, axis=0)[None, :]
        ss4_ref[...] += jnp.sum(y4 * y4, axis=0)[None, :]

    return pl.pallas_call(
        body,
        grid=(_NT,),
        in_specs=[
            _row_spec(100), _row_spec(128),
            _const_spec((1, 100)), _const_spec((1, 100)),
            _const_spec((228, 228)), _const_spec((1, 228)),
            _const_spec((1, 100)), _const_spec((1, 100)),
            _const_spec((128, 1024)), _const_spec((100, 1024)),
            _const_spec((228, 1024)), _const_spec((1, 1024)),
            _const_spec((1, 1024)), _const_spec((1, 1024)),
            _const_spec((1024, 512)), _const_spec((1, 512)),
        ],
        out_specs=[_row_spec(512), _const_spec((1, 512)),
                   _const_spec((1, 512)), _const_spec((1, 1024)),
                   _const_spec((1, 1024))],
        out_shape=[
            jax.ShapeDtypeStruct((_B, 512), _bf16),
            jax.ShapeDtypeStruct((1, 512), _f32),
            jax.ShapeDtypeStruct((1, 512), _f32),
            jax.ShapeDtypeStruct((1, 1024), _f32),
            jax.ShapeDtypeStruct((1, 1024), _f32),
        ],
        compiler_params=_SEQ,
    )(h2, x, s2, ss2, m2, sv2, g2, be2, w2at_x, w2at_h, w2at_f, b2a,
      g3, be3, w2bt, b2b)


def _p3(h2, x, s2, ss2, g2, be2, w2at_x, w2at_h, b2a):
    def body(h_ref, x_ref, s_ref, ss_ref, g_ref, be_ref, wx_ref, wh_ref,
             b_ref, y3_ref, s3_ref, ss3_ref):
        i = pl.program_id(0)
        scale, shift = _bn_coefs(s_ref[...], ss_ref[...], g_ref[...], be_ref[...])
        z = h_ref[...].astype(_f32) * scale + shift
        y = jnp.dot(x_ref[...].astype(_bf16), wx_ref[...],
                    preferred_element_type=_f32)
        y += jnp.dot(z.astype(_bf16), wh_ref[...], preferred_element_type=_f32)
        y += b_ref[...]
        y3_ref[...] = y.astype(_bf16)

        @pl.when(i == 0)
        def _():
            s3_ref[...] = jnp.zeros_like(s3_ref)
            ss3_ref[...] = jnp.zeros_like(ss3_ref)

        s3_ref[...] += jnp.sum(y, axis=0)[None, :]
        ss3_ref[...] += jnp.sum(y * y, axis=0)[None, :]

    return pl.pallas_call(
        body,
        grid=(_NT,),
        in_specs=[
            _row_spec(100), _row_spec(128),
            _const_spec((1, 100)), _const_spec((1, 100)),
            _const_spec((1, 100)), _const_spec((1, 100)),
            _const_spec((128, 1024)), _const_spec((100, 1024)),
            _const_spec((1, 1024)),
        ],
        out_specs=[_row_spec(1024), _const_spec((1, 1024)),
                   _const_spec((1, 1024))],
        out_shape=[
            jax.ShapeDtypeStruct((_B, 1024), _bf16),
            jax.ShapeDtypeStruct((1, 1024), _f32),
            jax.ShapeDtypeStruct((1, 1024), _f32),
        ],
        compiler_params=_SEQ,
    )(h2, x, s2, ss2, g2, be2, w2at_x, w2at_h, b2a)


def _p4(y3, s3, ss3, g3, be3, w2bt, b2b):
    def body(y3_ref, s_ref, ss_ref, g_ref, be_ref, w_ref, b_ref,
             y4_ref, s4_ref, ss4_ref):
        i = pl.program_id(0)
        scale, shift = _bn_coefs(s_ref[...], ss_ref[...], g_ref[...], be_ref[...])
        r = jnp.maximum(y3_ref[...].astype(_f32) * scale + shift, 0.0)
        y = jnp.dot(r.astype(_bf16), w_ref[...], preferred_element_type=_f32)
        y += b_ref[...]
        y4_ref[...] = y.astype(_bf16)

        @pl.when(i == 0)
        def _():
            s4_ref[...] = jnp.zeros_like(s4_ref)
            ss4_ref[...] = jnp.zeros_like(ss4_ref)

        s4_ref[...] += jnp.sum(y, axis=0)[None, :]
        ss4_ref[...] += jnp.sum(y * y, axis=0)[None, :]

    return pl.pallas_call(
        body,
        grid=(_NT,),
        in_specs=[
            _row_spec(1024),
            _const_spec((1, 1024)), _const_spec((1, 1024)),
            _const_spec((1, 1024)), _const_spec((1, 1024)),
            _const_spec((1024, 512)), _const_spec((1, 512)),
        ],
        out_specs=[_row_spec(512), _const_spec((1, 512)),
                   _const_spec((1, 512))],
        out_shape=[
            jax.ShapeDtypeStruct((_B, 512), _bf16),
            jax.ShapeDtypeStruct((1, 512), _f32),
            jax.ShapeDtypeStruct((1, 512), _f32),
        ],
        compiler_params=_SEQ,
    )(y3, s3, ss3, g3, be3, w2bt, b2b)


def _p5(y4, s4, ss4, g4, be4, w2c, b2c):
    def body(y4_ref, s_ref, ss_ref, g_ref, be_ref, w_ref, b_ref, o_ref):
        scale, shift = _bn_coefs(s_ref[...], ss_ref[...], g_ref[...], be_ref[...])
        r = jnp.maximum(y4_ref[...].astype(_f32) * scale + shift, 0.0)
        o = jnp.sum(r * w_ref[...], axis=1, keepdims=True) + b_ref[0, 0]
        o_ref[...] = o

    return pl.pallas_call(
        body,
        grid=(_NT,),
        in_specs=[
            _row_spec(512),
            _const_spec((1, 512)), _const_spec((1, 512)),
            _const_spec((1, 512)), _const_spec((1, 512)),
            _const_spec((1, 512)), _const_spec((1, 1)),
        ],
        out_specs=[_row_spec(1)],
        out_shape=[jax.ShapeDtypeStruct((_B, 1), _f32)],
        compiler_params=_SEQ,
    )(y4, s4, ss4, g4, be4, w2c, b2c)[0]


def kernel(x, pref, city, district, station,
           emb_pref, emb_city, emb_district, emb_station,
           W1a, b1a, a1, g1, be1,
           W1b, b1b, a2, g2, be2,
           W2a, b2a, g3, be3,
           W2b, b2b, g4, be4,
           W2c, b2c):
    city3 = city.reshape(_NW, _C_NCH, _C_CH)
    station3 = station.reshape(_NW, _C_NCH, _C_CH)
    district4 = district.reshape(_NCK, _NW, _D_NCH_K, _D_CH)

    gc, gs = _sc_gather_cs(_pad_cols(emb_city, 128),
                           _pad_cols(emb_station, 128),
                           city3, station3)
    ed_p = _transpose_pad(emb_district.T, 1024)
    gds = [_sc_gather_d(ed_p, district4[k]) for k in range(_NCK)]

    wp = W1a[:, 0:10].T.astype(_bf16)
    wc = jnp.pad(W1a[:, 10:110].T, ((0, 28), (0, 0))).astype(_bf16)
    wd = jnp.pad(W1a[:, 110:1110].T, ((0, 24), (0, 0))).astype(_bf16)
    ws = jnp.pad(W1a[:, 1110:1210].T, ((0, 28), (0, 0))).astype(_bf16)
    pref3 = pref.reshape(_B // _TB1, 1, _TB1)
    ep16 = emb_pref.astype(_bf16)
    b1a2 = b1a.reshape(1, 1000)
    a12 = a1.reshape(1, 1)

    h1 = None
    sparts, ssparts = [], []
    for k in range(_NCK):
        h1, sk, ssk = _p1(gds[k], gc, gs, pref3, ep16, wd, wc, ws, wp,
                          b1a2, a12, k, h1)
        sparts.append(sk)
        ssparts.append(ssk)
    s1 = jnp.concatenate(sparts, axis=0)
    ss1 = jnp.concatenate(ssparts, axis=0)

    h2, s2, ss2, m2, sv2 = _p2(h1, s1, ss1, x, g1.reshape(1, 1000),
                               be1.reshape(1, 1000), W1b.T.astype(_bf16),
                               b1b.reshape(1, 100), a2.reshape(1, 1))

    w2at_f = W2a.T
    w2at = w2at_f.astype(_bf16)
    y4, s4, ss4, _, _ = _p34(h2, x, s2, ss2, m2, sv2, g2.reshape(1, 100),
                             be2.reshape(1, 100), w2at[0:128], w2at[128:228],
                             w2at_f, b2a.reshape(1, 1024),
                             g3.reshape(1, 1024), be3.reshape(1, 1024),
                             W2b.T.astype(_bf16), b2b.reshape(1, 512))

    return _p5(y4, s4, ss4, g4.reshape(1, 512), be4.reshape(1, 512),
               W2c.reshape(1, 512), b2c.reshape(1, 1))
